# baseline (speedup 1.0000x reference)
import functools

import jax
import jax.numpy as jnp
from jax import lax
from jax.experimental import pallas as pl
from jax.experimental.pallas import tpu as pltpu
from jax.experimental.pallas import tpu_sc as plsc

N = 10000
E = 320000
D = 128
EPS = 1e-5

NC = 2
NS = 16
L = 16

NP8 = 10240
RPT = NP8 // NS
CHUNK = 64
EPT = 10112
NCH = EPT // CHUNK
EP = NC * NS * EPT
NBUF = 4
DUMMY = N + 16

_MESH = plsc.VectorSubcoreMesh(
    core_axis_name="c", subcore_axis_name="s", num_cores=NC, num_subcores=NS)


def _zero_fill(buf, nrows):
    def fz(i, carry):
        for t in range(D // L):
            buf[i, pl.ds(t * L, L)] = jnp.zeros((L,), jnp.float32)
        return carry
    lax.fori_loop(0, nrows, fz, 0)


@functools.partial(
    pl.kernel,
    out_type=jax.ShapeDtypeStruct((NC, NP8, D), jnp.float32),
    mesh=_MESH,
    scratch_types=[
        pltpu.VMEM_SHARED((NP8, D), jnp.float32),
        pltpu.VMEM((CHUNK, D), jnp.float32),
    ] + [pltpu.VMEM((CHUNK,), jnp.int32)] * NBUF
      + [pltpu.SemaphoreType.DMA] * NBUF
      + [pltpu.SemaphoreType.DMA] * NBUF,
)
def _deg_kernel(dst_hbm, out_hbm, acc, ones_v,
                d0, d1, d2, d3, dm0, dm1, dm2, dm3, s0, s1, s2, s3):
    c = lax.axis_index("c")
    s = lax.axis_index("s")
    db = (d0, d1, d2, d3)
    dsem = (dm0, dm1, dm2, dm3)
    ssem = (s0, s1, s2, s3)

    _zero_fill(ones_v, CHUNK)
    for t in range(RPT // CHUNK):
        pltpu.sync_copy(ones_v, acc.at[pl.ds(s * RPT + t * CHUNK, CHUNK)])

    def fo(i, carry):
        for t in range(D // L):
            ones_v[i, pl.ds(t * L, L)] = jnp.zeros((L,), jnp.float32) + 1.0
        return carry
    lax.fori_loop(0, CHUNK, fo, 0)
    plsc.subcore_barrier()

    base = (c * NS + s) * EPT
    for b in range(NBUF - 1):
        pltpu.async_copy(dst_hbm.at[pl.ds(base + b * CHUNK, CHUNK)],
                         db[b], dsem[b])

    def quad(q, carry):
        for b in range(NBUF):
            t = q * NBUF + b
            bp = (b + NBUF - 1) % NBUF

            @pl.when(t < NCH)
            def _body():
                pltpu.make_async_copy(
                    dst_hbm.at[pl.ds(base + t * CHUNK, CHUNK)],
                    db[b], dsem[b]).wait()
                pltpu.async_copy(ones_v, acc.at[db[b]], ssem[b], add=True)
                tp = t + NBUF - 1

                @pl.when(tp < NCH)
                def _pf():
                    @pl.when(t >= 1)
                    def _ws():
                        pltpu.make_async_copy(
                            ones_v, acc.at[db[bp]], ssem[bp]).wait()
                    pltpu.async_copy(
                        dst_hbm.at[pl.ds(base + tp * CHUNK, CHUNK)],
                        db[bp], dsem[bp])
        return carry

    lax.fori_loop(0, (NCH + NBUF - 1) // NBUF, quad, 0)
    for b in range(NBUF):
        t = NCH - NBUF + b
        pltpu.make_async_copy(ones_v, acc.at[db[t % NBUF]],
                              ssem[t % NBUF]).wait()
    plsc.subcore_barrier()
    pltpu.sync_copy(acc.at[pl.ds(s * RPT, RPT)],
                    out_hbm.at[c, pl.ds(s * RPT, RPT)])


@functools.partial(
    pl.kernel,
    out_type=jax.ShapeDtypeStruct((NC, NP8, D), jnp.float32),
    mesh=_MESH,
    scratch_types=[
        pltpu.VMEM_SHARED((NP8, D), jnp.float32),
        pltpu.VMEM((EPT,), jnp.int32),
    ] + [pltpu.VMEM((CHUNK, D), jnp.float32)] * NBUF
      + [pltpu.VMEM((CHUNK,), jnp.int32)] * NBUF
      + [pltpu.SemaphoreType.DMA] * (3 * NBUF),
)
def _scatter_kernel(y_hbm, src_hbm, dst_hbm, out_hbm, acc, src_sl,
                    r0, r1, r2, r3, d0, d1, d2, d3,
                    g0, g1, g2, g3, dm0, dm1, dm2, dm3, s0, s1, s2, s3):
    c = lax.axis_index("c")
    s = lax.axis_index("s")
    rows = (r0, r1, r2, r3)
    db = (d0, d1, d2, d3)
    gsem = (g0, g1, g2, g3)
    dsem = (dm0, dm1, dm2, dm3)
    ssem = (s0, s1, s2, s3)

    _zero_fill(r0, CHUNK)
    for t in range(RPT // CHUNK):
        pltpu.sync_copy(r0, acc.at[pl.ds(s * RPT + t * CHUNK, CHUNK)])

    base = (c * NS + s) * EPT
    pltpu.sync_copy(src_hbm.at[pl.ds(base, EPT)], src_sl)
    plsc.subcore_barrier()

    for b in range(NBUF - 1):
        pltpu.async_copy(dst_hbm.at[pl.ds(base + b * CHUNK, CHUNK)],
                         db[b], dsem[b])
        pltpu.async_copy(y_hbm.at[src_sl.at[pl.ds(b * CHUNK, CHUNK)]],
                         rows[b], gsem[b])

    def quad(q, carry):
        for b in range(NBUF):
            t = q * NBUF + b
            bp = (b + NBUF - 1) % NBUF

            @pl.when(t < NCH)
            def _body():
                pltpu.make_async_copy(
                    dst_hbm.at[pl.ds(base + t * CHUNK, CHUNK)],
                    db[b], dsem[b]).wait()
                pltpu.make_async_copy(
                    y_hbm.at[src_sl.at[pl.ds(t * CHUNK, CHUNK)]],
                    rows[b], gsem[b]).wait()
                pltpu.async_copy(rows[b], acc.at[db[b]], ssem[b], add=True)
                tp = t + NBUF - 1

                @pl.when(tp < NCH)
                def _pf():
                    @pl.when(t >= 1)
                    def _ws():
                        pltpu.make_async_copy(
                            rows[bp], acc.at[db[bp]], ssem[bp]).wait()
                    pltpu.async_copy(
                        dst_hbm.at[pl.ds(base + tp * CHUNK, CHUNK)],
                        db[bp], dsem[bp])
                    pltpu.async_copy(
                        y_hbm.at[src_sl.at[pl.ds(tp * CHUNK, CHUNK)]],
                        rows[bp], gsem[bp])
        return carry

    lax.fori_loop(0, (NCH + NBUF - 1) // NBUF, quad, 0)
    for b in range(NBUF):
        t = NCH - NBUF + b
        pltpu.make_async_copy(rows[t % NBUF], acc.at[db[t % NBUF]],
                              ssem[t % NBUF]).wait()
    plsc.subcore_barrier()
    pltpu.sync_copy(acc.at[pl.ds(s * RPT, RPT)],
                    out_hbm.at[c, pl.ds(s * RPT, RPT)])


BN = 2000


def _prep_body(deg_ref, x_ref, dinv_ref, y_ref):
    deg = deg_ref[0, :, 0:1] + deg_ref[1, :, 0:1] + 1.0
    dinv = lax.rsqrt(jnp.maximum(deg, 1.0))
    dinv_ref[...] = dinv
    y_ref[...] = x_ref[...] * dinv


def _layer_body(acc_ref, y_ref, dinv_ref, w_ref, b_ref, g_ref, beta_ref,
                out_ref):
    dinv = dinv_ref[...]
    agg = (y_ref[...] + acc_ref[0] + acc_ref[1]) * dinv
    h = jnp.dot(agg, w_ref[...], preferred_element_type=jnp.float32)
    h = jnp.maximum(h + b_ref[...], 0.0)
    mu = jnp.mean(h, axis=1, keepdims=True)
    var = jnp.mean((h - mu) * (h - mu), axis=1, keepdims=True)
    ln = (h - mu) * lax.rsqrt(var + EPS) * g_ref[...] + beta_ref[...]
    out_ref[...] = ln * dinv


def _final_body(acc_ref, y_ref, dinv_ref, w_ref, b_ref, g_ref, beta_ref,
                wout_ref, out_ref):
    dinv = dinv_ref[...]
    agg = (y_ref[...] + acc_ref[0] + acc_ref[1]) * dinv
    h = jnp.dot(agg, w_ref[...], preferred_element_type=jnp.float32)
    h = jnp.maximum(h + b_ref[...], 0.0)
    mu = jnp.mean(h, axis=1, keepdims=True)
    var = jnp.mean((h - mu) * (h - mu), axis=1, keepdims=True)
    ln = (h - mu) * lax.rsqrt(var + EPS) * g_ref[...] + beta_ref[...]
    out_ref[...] = jnp.dot(ln, wout_ref[...],
                           preferred_element_type=jnp.float32)


_ROWS = pl.BlockSpec((BN, D), lambda i: (i, 0))
_ACC = pl.BlockSpec((NC, BN, D), lambda i: (0, i, 0))
_DINV = pl.BlockSpec((BN, 1), lambda i: (i, 0))
_MAT = pl.BlockSpec((D, D), lambda i: (0, 0))
_VEC = pl.BlockSpec((1, D), lambda i: (0, 0))

_prep_call = pl.pallas_call(
    _prep_body,
    grid=(N // BN,),
    in_specs=[_ACC, _ROWS],
    out_specs=[_DINV, _ROWS],
    out_shape=[jax.ShapeDtypeStruct((N, 1), jnp.float32),
               jax.ShapeDtypeStruct((N, D), jnp.float32)],
)

_layer_call = pl.pallas_call(
    _layer_body,
    grid=(N // BN,),
    in_specs=[_ACC, _ROWS, _DINV, _MAT, _VEC, _VEC, _VEC],
    out_specs=_ROWS,
    out_shape=jax.ShapeDtypeStruct((N, D), jnp.float32),
)

_final_call = pl.pallas_call(
    _final_body,
    grid=(N // BN,),
    in_specs=[_ACC, _ROWS, _DINV, _MAT, _VEC, _VEC, _VEC, _MAT],
    out_specs=_ROWS,
    out_shape=jax.ShapeDtypeStruct((N, D), jnp.float32),
)


def _pad_edges(idx, fill):
    per = E // (NC * NS)
    x2 = idx.reshape(NC * NS, per)
    return jnp.pad(x2, ((0, 0), (0, EPT - per)),
                   constant_values=fill).reshape(EP)


def kernel(x, edge_index, W0, b0, g0, beta0, W1, b1, g1, beta1,
           W2, b2, g2, beta2, W_out):
    src = _pad_edges(edge_index[0], 0)
    dst = _pad_edges(edge_index[1], DUMMY)
    deg2 = _deg_kernel(dst)
    dinv, y = _prep_call(deg2, x)
    params = [(W0, b0, g0, beta0), (W1, b1, g1, beta1), (W2, b2, g2, beta2)]
    for k, (W, b, g, beta) in enumerate(params):
        acc2 = _scatter_kernel(y, src, dst)
        bv = b.reshape(1, D)
        gv = g.reshape(1, D)
        betav = beta.reshape(1, D)
        if k < 2:
            y = _layer_call(acc2, y, dinv, W, bv, gv, betav)
        else:
            out = _final_call(acc2, y, dinv, W, bv, gv, betav, W_out)
    return out


# speedup vs baseline: 1.9285x; 1.9285x over previous
import functools

import jax
import jax.numpy as jnp
from jax import lax
from jax.experimental import pallas as pl
from jax.experimental.pallas import tpu as pltpu
from jax.experimental.pallas import tpu_sc as plsc

N = 10000
E = 320000
D = 128
EPS = 1e-5

NC = 2
NS = 16
L = 16

NP8 = 10240
RPT = NP8 // NS
NBUF = 4

CHUNK = 80
EPT = E // (NC * NS)
NCH = EPT // CHUNK

CHD = 128
EPTD = 10112
NCHD = EPTD // CHD
EP = NC * NS * EPTD
DUMMY = N + 16

_MESH = plsc.VectorSubcoreMesh(
    core_axis_name="c", subcore_axis_name="s", num_cores=NC, num_subcores=NS)


def _zero_fill(buf, nrows):
    def fz(i, carry):
        for t in range(D // L):
            buf[i, pl.ds(t * L, L)] = jnp.zeros((L,), jnp.float32)
        return carry
    lax.fori_loop(0, nrows, fz, 0)


@functools.partial(
    pl.kernel,
    out_type=jax.ShapeDtypeStruct((NC, NP8, D), jnp.float32),
    mesh=_MESH,
    scratch_types=[
        pltpu.VMEM_SHARED((NP8, D), jnp.float32),
        pltpu.VMEM((CHD, D), jnp.float32),
    ] + [pltpu.VMEM((CHD,), jnp.int32)] * NBUF
      + [pltpu.SemaphoreType.DMA] * NBUF
      + [pltpu.SemaphoreType.DMA] * NBUF,
)
def _deg_kernel(dst_hbm, out_hbm, acc, ones_v,
                d0, d1, d2, d3, dm0, dm1, dm2, dm3, s0, s1, s2, s3):
    c = lax.axis_index("c")
    s = lax.axis_index("s")
    db = (d0, d1, d2, d3)
    dsem = (dm0, dm1, dm2, dm3)
    ssem = (s0, s1, s2, s3)

    _zero_fill(ones_v, CHD)
    for t in range(RPT // CHD):
        pltpu.sync_copy(ones_v, acc.at[pl.ds(s * RPT + t * CHD, CHD)])

    def fo(i, carry):
        for t in range(D // L):
            ones_v[i, pl.ds(t * L, L)] = jnp.zeros((L,), jnp.float32) + 1.0
        return carry
    lax.fori_loop(0, CHD, fo, 0)
    plsc.subcore_barrier()

    base = (c * NS + s) * EPTD
    for b in range(NBUF - 1):
        pltpu.async_copy(dst_hbm.at[pl.ds(base + b * CHD, CHD)],
                         db[b], dsem[b])

    def quad(q, carry):
        for b in range(NBUF):
            t = q * NBUF + b
            bp = (b + NBUF - 1) % NBUF

            @pl.when(t < NCHD)
            def _body():
                pltpu.make_async_copy(
                    dst_hbm.at[pl.ds(base + t * CHD, CHD)],
                    db[b], dsem[b]).wait()
                pltpu.async_copy(ones_v, acc.at[db[b]], ssem[b], add=True)
                tp = t + NBUF - 1

                @pl.when(tp < NCHD)
                def _pf():
                    @pl.when(t >= 1)
                    def _ws():
                        pltpu.make_async_copy(
                            ones_v, acc.at[db[bp]], ssem[bp]).wait()
                    pltpu.async_copy(
                        dst_hbm.at[pl.ds(base + tp * CHD, CHD)],
                        db[bp], dsem[bp])
        return carry

    lax.fori_loop(0, (NCHD + NBUF - 1) // NBUF, quad, 0)
    for b in range(NBUF):
        t = NCHD - NBUF + b
        pltpu.make_async_copy(ones_v, acc.at[db[t % NBUF]],
                              ssem[t % NBUF]).wait()
    plsc.subcore_barrier()
    pltpu.sync_copy(acc.at[pl.ds(s * RPT, RPT)],
                    out_hbm.at[c, pl.ds(s * RPT, RPT)])


@functools.partial(
    pl.kernel,
    out_type=jax.ShapeDtypeStruct((NC, NP8, D), jnp.float32),
    mesh=_MESH,
    scratch_types=[
        pltpu.VMEM_SHARED((NP8, D), jnp.float32),
    ] + [pltpu.VMEM((CHUNK, D), jnp.float32)] * NBUF
      + [pltpu.VMEM((CHUNK,), jnp.int32)] * (2 * NBUF)
      + [pltpu.SemaphoreType.DMA] * (4 * NBUF),
)
def _scatter_kernel(y_hbm, src_hbm, dst_hbm, out_hbm, acc,
                    r0, r1, r2, r3, e0, e1, e2, e3, d0, d1, d2, d3,
                    g0, g1, g2, g3, em0, em1, em2, em3,
                    dm0, dm1, dm2, dm3, s0, s1, s2, s3):
    c = lax.axis_index("c")
    s = lax.axis_index("s")
    rows = (r0, r1, r2, r3)
    sb = (e0, e1, e2, e3)
    db = (d0, d1, d2, d3)
    gsem = (g0, g1, g2, g3)
    esem = (em0, em1, em2, em3)
    dsem = (dm0, dm1, dm2, dm3)
    ssem = (s0, s1, s2, s3)

    _zero_fill(r0, CHUNK)
    for t in range(RPT // CHUNK):
        pltpu.sync_copy(r0, acc.at[pl.ds(s * RPT + t * CHUNK, CHUNK)])
    plsc.subcore_barrier()

    base = (c * NS + s) * EPT
    for b in range(NBUF - 1):
        off = base + b * CHUNK
        pltpu.async_copy(src_hbm.at[pl.ds(off, CHUNK)], sb[b], esem[b])
        pltpu.async_copy(dst_hbm.at[pl.ds(off, CHUNK)], db[b], dsem[b])
    for b in range(NBUF - 2):
        off = base + b * CHUNK
        pltpu.make_async_copy(src_hbm.at[pl.ds(off, CHUNK)], sb[b],
                              esem[b]).wait()
        pltpu.async_copy(y_hbm.at[sb[b]], rows[b], gsem[b])

    def quad(q, carry):
        for b in range(NBUF):
            t = q * NBUF + b
            bp = (b + NBUF - 1) % NBUF
            bg = (b + NBUF - 2) % NBUF

            @pl.when(t < NCH)
            def _body():
                pltpu.make_async_copy(
                    dst_hbm.at[pl.ds(base + t * CHUNK, CHUNK)],
                    db[b], dsem[b]).wait()
                pltpu.make_async_copy(y_hbm.at[sb[b]], rows[b],
                                      gsem[b]).wait()
                pltpu.async_copy(rows[b], acc.at[db[b]], ssem[b], add=True)
                tp = t + NBUF - 1

                @pl.when(tp < NCH)
                def _pf():
                    @pl.when(t >= 1)
                    def _ws():
                        pltpu.make_async_copy(
                            rows[bp], acc.at[db[bp]], ssem[bp]).wait()
                    off = base + tp * CHUNK
                    pltpu.async_copy(src_hbm.at[pl.ds(off, CHUNK)],
                                     sb[bp], esem[bp])
                    pltpu.async_copy(dst_hbm.at[pl.ds(off, CHUNK)],
                                     db[bp], dsem[bp])
                tg = t + NBUF - 2

                @pl.when(tg < NCH)
                def _pg():
                    pltpu.make_async_copy(
                        src_hbm.at[pl.ds(base + tg * CHUNK, CHUNK)],
                        sb[bg], esem[bg]).wait()
                    pltpu.async_copy(y_hbm.at[sb[bg]], rows[bg], gsem[bg])
        return carry

    lax.fori_loop(0, (NCH + NBUF - 1) // NBUF, quad, 0)
    for b in range(NBUF):
        t = NCH - NBUF + b
        pltpu.make_async_copy(rows[t % NBUF], acc.at[db[t % NBUF]],
                              ssem[t % NBUF]).wait()
    plsc.subcore_barrier()
    pltpu.sync_copy(acc.at[pl.ds(s * RPT, RPT)],
                    out_hbm.at[c, pl.ds(s * RPT, RPT)])


BN = 2000


def _prep_body(deg_ref, x_ref, dinv_ref, y_ref):
    deg = deg_ref[0, :, 0:1] + deg_ref[1, :, 0:1] + 1.0
    dinv = lax.rsqrt(jnp.maximum(deg, 1.0))
    dinv_ref[...] = dinv
    y_ref[...] = x_ref[...] * dinv


def _layer_body(acc_ref, y_ref, dinv_ref, w_ref, b_ref, g_ref, beta_ref,
                out_ref):
    dinv = dinv_ref[...]
    agg = (y_ref[...] + acc_ref[0] + acc_ref[1]) * dinv
    h = jnp.dot(agg, w_ref[...], preferred_element_type=jnp.float32)
    h = jnp.maximum(h + b_ref[...], 0.0)
    mu = jnp.mean(h, axis=1, keepdims=True)
    var = jnp.mean((h - mu) * (h - mu), axis=1, keepdims=True)
    ln = (h - mu) * lax.rsqrt(var + EPS) * g_ref[...] + beta_ref[...]
    out_ref[...] = ln * dinv


def _final_body(acc_ref, y_ref, dinv_ref, w_ref, b_ref, g_ref, beta_ref,
                wout_ref, out_ref):
    dinv = dinv_ref[...]
    agg = (y_ref[...] + acc_ref[0] + acc_ref[1]) * dinv
    h = jnp.dot(agg, w_ref[...], preferred_element_type=jnp.float32)
    h = jnp.maximum(h + b_ref[...], 0.0)
    mu = jnp.mean(h, axis=1, keepdims=True)
    var = jnp.mean((h - mu) * (h - mu), axis=1, keepdims=True)
    ln = (h - mu) * lax.rsqrt(var + EPS) * g_ref[...] + beta_ref[...]
    out_ref[...] = jnp.dot(ln, wout_ref[...],
                           preferred_element_type=jnp.float32)


_ROWS = pl.BlockSpec((BN, D), lambda i: (i, 0))
_ACC = pl.BlockSpec((NC, BN, D), lambda i: (0, i, 0))
_DINV = pl.BlockSpec((BN, 1), lambda i: (i, 0))
_MAT = pl.BlockSpec((D, D), lambda i: (0, 0))
_VEC = pl.BlockSpec((1, D), lambda i: (0, 0))

_prep_call = pl.pallas_call(
    _prep_body,
    grid=(N // BN,),
    in_specs=[_ACC, _ROWS],
    out_specs=[_DINV, _ROWS],
    out_shape=[jax.ShapeDtypeStruct((N, 1), jnp.float32),
               jax.ShapeDtypeStruct((N, D), jnp.float32)],
)

_layer_call = pl.pallas_call(
    _layer_body,
    grid=(N // BN,),
    in_specs=[_ACC, _ROWS, _DINV, _MAT, _VEC, _VEC, _VEC],
    out_specs=_ROWS,
    out_shape=jax.ShapeDtypeStruct((N, D), jnp.float32),
)

_final_call = pl.pallas_call(
    _final_body,
    grid=(N // BN,),
    in_specs=[_ACC, _ROWS, _DINV, _MAT, _VEC, _VEC, _VEC, _MAT],
    out_specs=_ROWS,
    out_shape=jax.ShapeDtypeStruct((N, D), jnp.float32),
)


def kernel(x, edge_index, W0, b0, g0, beta0, W1, b1, g1, beta1,
           W2, b2, g2, beta2, W_out):
    src = edge_index[0]
    dst = edge_index[1]
    dstp = jnp.pad(dst.reshape(NC * NS, EPT), ((0, 0), (0, EPTD - EPT)),
                   constant_values=DUMMY).reshape(EP)
    deg2 = _deg_kernel(dstp)
    dinv, y = _prep_call(deg2, x)
    params = [(W0, b0, g0, beta0), (W1, b1, g1, beta1), (W2, b2, g2, beta2)]
    for k, (W, b, g, beta) in enumerate(params):
        acc2 = _scatter_kernel(y, src, dst)
        bv = b.reshape(1, D)
        gv = g.reshape(1, D)
        betav = beta.reshape(1, D)
        if k < 2:
            y = _layer_call(acc2, y, dinv, W, bv, gv, betav)
        else:
            out = _final_call(acc2, y, dinv, W, bv, gv, betav, W_out)
    return out
